# baseline (device time: 195818 ns/iter reference)
import os

import jax
import jax.numpy as jnp
from jax import lax
from jax.experimental import pallas as pl
from jax.experimental.pallas import tpu as pltpu

N_DEV = 4
N_HOPS = N_DEV - 1
N_HEADS = 16
HEAD_DIM = 128
H_HALF = N_HEADS // 2
SCALE = HEAD_DIM**-0.5 * 1.4426950408889634
_NO_COMM = bool(os.environ.get("NO_COMM"))
_NO_COMPUTE = bool(os.environ.get("NO_COMPUTE"))


def _body(
    q_hbm, k_hbm, v_hbm, out_ref, commk_ref, commv_ref, m_ref, l_ref,
    q_ref, k_ref, v_ref, stage_ref, send_sems, recv_sems, local_sems,
):
    my = lax.axis_index("i")
    left = lax.rem(my + N_DEV - 1, N_DEV)
    right = lax.rem(my + 1, N_DEV)

    barrier = pltpu.get_barrier_semaphore()
    for nbr in (left, right):
        pl.semaphore_signal(
            barrier, inc=1, device_id=(nbr,), device_id_type=pl.DeviceIdType.MESH
        )
    pl.semaphore_wait(barrier, 2)

    half = H_HALF * HEAD_DIM
    seq_q = out_ref.shape[0]
    hp = seq_q // 2

    def piece_copy(src_hbm, piece, slot, sem_idx):
        rows = slice(piece * hp, (piece + 1) * hp)
        return pltpu.make_async_copy(
            src_hbm.at[rows], stage_ref.at[slot], local_sems.at[sem_idx]
        ), rows

    def cast_piece(dst_ref, rows, slot, scale=None):
        x = stage_ref[slot]
        if scale is not None:
            x = x * scale
        dst_ref[rows] = x.astype(jnp.bfloat16)

    cps = {}
    for i, (name, src) in enumerate((("k", k_hbm), ("v", v_hbm))):
        for piece in (0, 1):
            cp, rows = piece_copy(src, piece, piece, 2 * i + piece)
            cps[(name, piece)] = (cp, rows)
    cps[("k", 0)][0].start()
    cps[("k", 1)][0].start()
    for name, dst in (("k", k_ref), ("v", v_ref)):
        for piece in (0, 1):
            cp, rows = cps[(name, piece)]
            cp.wait()
            cast_piece(dst, rows, piece)
            if name == "k":
                cps[("v", piece)][0].start()

    def hop_rdmas(c, rows=slice(None), idx_base=None):
        base = 4 * c if idx_base is None else idx_base
        rs = []
        for d, tgt in ((0, right), (1, left)):
            ls = slice(0, half) if d == 0 else slice(half, 2 * half)
            for t, (in_ref, comm) in enumerate(
                ((k_ref, commk_ref), (v_ref, commv_ref))
            ):
                src = in_ref.at[rows, ls] if c == 0 else comm.at[c - 1, rows, ls]
                idx = base + d * 2 + t
                rs.append(
                    pltpu.make_async_remote_copy(
                        src_ref=src,
                        dst_ref=comm.at[c, rows, ls],
                        send_sem=send_sems.at[idx],
                        recv_sem=recv_sems.at[idx],
                        device_id=(tgt,),
                        device_id_type=pl.DeviceIdType.MESH,
                    )
                )
        return rs

    def compute_phase(c, rows=slice(None), finalize=None):
        if finalize is None:
            finalize = c == N_DEV - 1

        def head_body(head, carry):
            lane = pl.ds(head * HEAD_DIM, HEAD_DIM)
            q_h = q_ref[:, lane]
            if c == 0:
                k_c, v_c = k_ref[rows, lane], v_ref[rows, lane]
            else:
                k_c = commk_ref[c - 1, rows, lane]
                v_c = commv_ref[c - 1, rows, lane]
            s = lax.dot_general(
                q_h, k_c, (((1,), (1,)), ((), ())),
                preferred_element_type=jnp.float32,
            )
            hmask = (
                lax.broadcasted_iota(jnp.int32, (seq_q, N_HEADS), 1) == head
            )
            if c == 0:
                m = jnp.max(s, axis=1, keepdims=True)
                m_ref[...] = jnp.where(hmask, m, m_ref[...])
            else:
                m = jnp.sum(
                    jnp.where(hmask, m_ref[...], 0.0), axis=1, keepdims=True
                )
            p = jnp.exp2(s - m)
            l_c = jnp.sum(p, axis=1, keepdims=True)
            pv = lax.dot_general(
                p.astype(jnp.bfloat16), v_c,
                (((1,), (0,)), ((), ())),
                preferred_element_type=jnp.float32,
            )
            if c == 0:
                l_new = l_c
                acc = pv
            else:
                l_old = jnp.sum(
                    jnp.where(hmask, l_ref[...], 0.0), axis=1, keepdims=True
                )
                l_new = l_old + l_c
                acc = out_ref[:, lane] + pv
            if finalize:
                out_ref[:, lane] = acc / l_new
            else:
                out_ref[:, lane] = acc
                l_ref[...] = jnp.where(hmask, l_new, l_ref[...])
            return carry

        if _NO_COMPUTE:
            if c == N_DEV - 1:
                out_ref[:, 0:half] = commk_ref[0, :, 0:half].astype(jnp.float32)
            return
        lax.fori_loop(0, N_HEADS, head_body, 0)

    def load_q():
        cq0, rows0 = piece_copy(q_hbm, 0, 0, 4)
        cq1, rows1 = piece_copy(q_hbm, 1, 1, 5)
        cq0.start()
        cq1.start()
        cq0.wait()
        cast_piece(q_ref, rows0, 0, scale=SCALE)
        cq1.wait()
        cast_piece(q_ref, rows1, 1, scale=SCALE)

    if _NO_COMM:
        load_q()
        for c in range(N_DEV):
            compute_phase(c)
        return
    all_rdmas = []
    prev = hop_rdmas(0)
    all_rdmas += prev
    for r in prev:
        r.start()
    load_q()
    compute_phase(0)
    for c in range(1, N_HOPS):
        for r in prev:
            r.wait_recv()
        if c < N_HOPS - 1:
            prev = hop_rdmas(c)
            all_rdmas += prev
            for r in prev:
                r.start()
        else:
            sub_a = hop_rdmas(c, rows=slice(0, hp), idx_base=4 * c)
            sub_b = hop_rdmas(c, rows=slice(hp, seq_q), idx_base=4 * (c + 1))
            all_rdmas += sub_a + sub_b
            for r in sub_a + sub_b:
                r.start()
        compute_phase(c)
    for r in sub_a:
        r.wait_recv()
    compute_phase(N_DEV - 1, rows=slice(0, hp), finalize=False)
    for r in sub_b:
        r.wait_recv()
    compute_phase(N_DEV - 1, rows=slice(hp, seq_q), finalize=True)
    for r in all_rdmas:
        r.wait_send()


def kernel(Q, K, V):
    b, s, h, d = Q.shape
    q = Q[0].reshape(s, h * d)
    k = K[0].reshape(s, h * d)
    v = V[0].reshape(s, h * d)

    out = pl.pallas_call(
        _body,
        out_shape=jax.ShapeDtypeStruct((s, h * d), jnp.float32),
        in_specs=[pl.BlockSpec(memory_space=pltpu.MemorySpace.HBM)] * 3,
        out_specs=pl.BlockSpec(memory_space=pltpu.VMEM),
        scratch_shapes=[
            pltpu.VMEM((N_HOPS, s, h * d), jnp.bfloat16),
            pltpu.VMEM((N_HOPS, s, h * d), jnp.bfloat16),
            pltpu.VMEM((s, N_HEADS), jnp.float32),
            pltpu.VMEM((s, N_HEADS), jnp.float32),
            pltpu.VMEM((s, h * d), jnp.bfloat16),
            pltpu.VMEM((s, h * d), jnp.bfloat16),
            pltpu.VMEM((s, h * d), jnp.bfloat16),
            pltpu.VMEM((2, s // 2, h * d), jnp.float32),
            pltpu.SemaphoreType.DMA((4 * (N_HOPS + 1),)),
            pltpu.SemaphoreType.DMA((4 * (N_HOPS + 1),)),
            pltpu.SemaphoreType.DMA((6,)),
        ],
        compiler_params=pltpu.CompilerParams(
            collective_id=0, vmem_limit_bytes=100 * 1024 * 1024
        ),
    )(q, k, v)
    return out.reshape(s, h, d)[None]


# device time: 190547 ns/iter; 1.0277x vs baseline; 1.0277x over previous
import os

import jax
import jax.numpy as jnp
from jax import lax
from jax.experimental import pallas as pl
from jax.experimental.pallas import tpu as pltpu

N_DEV = 4
N_HOPS = N_DEV - 1
N_HEADS = 16
HEAD_DIM = 128
H_HALF = N_HEADS // 2
_NO_COMM = bool(os.environ.get("NO_COMM"))
_NO_COMPUTE = bool(os.environ.get("NO_COMPUTE"))


def _body(
    q_ref, k_ref, v_ref, out_ref, commk_ref, commv_ref, m_ref, l_ref,
    send_sems, recv_sems,
):
    my = lax.axis_index("i")
    left = lax.rem(my + N_DEV - 1, N_DEV)
    right = lax.rem(my + 1, N_DEV)

    barrier = pltpu.get_barrier_semaphore()
    for nbr in (left, right):
        pl.semaphore_signal(
            barrier, inc=1, device_id=(nbr,), device_id_type=pl.DeviceIdType.MESH
        )
    pl.semaphore_wait(barrier, 2)

    half = H_HALF * HEAD_DIM
    seq_q = q_ref.shape[0]

    def hop_rdmas(c, rows=slice(None), idx_base=None):
        base = 4 * c if idx_base is None else idx_base
        rs = []
        for d, tgt in ((0, right), (1, left)):
            ls = slice(0, half) if d == 0 else slice(half, 2 * half)
            for t, (in_ref, comm) in enumerate(
                ((k_ref, commk_ref), (v_ref, commv_ref))
            ):
                src = in_ref.at[rows, ls] if c == 0 else comm.at[c - 1, rows, ls]
                idx = base + d * 2 + t
                rs.append(
                    pltpu.make_async_remote_copy(
                        src_ref=src,
                        dst_ref=comm.at[c, rows, ls],
                        send_sem=send_sems.at[idx],
                        recv_sem=recv_sems.at[idx],
                        device_id=(tgt,),
                        device_id_type=pl.DeviceIdType.MESH,
                    )
                )
        return rs

    def compute_phase(c, rows=slice(None), finalize=None):
        if finalize is None:
            finalize = c == N_DEV - 1

        def head_body(head, carry):
            lane = pl.ds(head * HEAD_DIM, HEAD_DIM)
            q_h = q_ref[:, lane]
            if c == 0:
                k_c, v_c = k_ref[rows, lane], v_ref[rows, lane]
            else:
                k_c = commk_ref[c - 1, rows, lane]
                v_c = commv_ref[c - 1, rows, lane]
            s = lax.dot_general(
                q_h, k_c, (((1,), (1,)), ((), ())),
                preferred_element_type=jnp.float32,
            )
            hmask = (
                lax.broadcasted_iota(jnp.int32, (seq_q, N_HEADS), 1) == head
            )
            if c == 0:
                m = jnp.max(s, axis=1, keepdims=True)
                m_ref[...] = jnp.where(hmask, m, m_ref[...])
            else:
                m = jnp.sum(
                    jnp.where(hmask, m_ref[...], 0.0), axis=1, keepdims=True
                )
            p = jnp.exp2(s - m)
            l_c = jnp.sum(p, axis=1, keepdims=True)
            pv = lax.dot_general(
                p.astype(jnp.bfloat16), v_c,
                (((1,), (0,)), ((), ())),
                preferred_element_type=jnp.float32,
            )
            if c == 0:
                l_new = l_c
                acc = pv
            else:
                l_old = jnp.sum(
                    jnp.where(hmask, l_ref[...], 0.0), axis=1, keepdims=True
                )
                l_new = l_old + l_c
                acc = out_ref[:, lane] + pv
            if finalize:
                out_ref[:, lane] = acc / l_new
            else:
                out_ref[:, lane] = acc
                l_ref[...] = jnp.where(hmask, l_new, l_ref[...])
            return carry

        if _NO_COMPUTE:
            if c == N_DEV - 1:
                out_ref[:, 0:half] = commk_ref[0, :, 0:half].astype(jnp.float32)
            return
        lax.fori_loop(0, N_HEADS, head_body, 0)

    if _NO_COMM:
        for c in range(N_DEV):
            compute_phase(c)
        return
    all_rdmas = []
    prev = hop_rdmas(0)
    all_rdmas += prev
    for r in prev:
        r.start()
    compute_phase(0)
    for c in range(1, N_HOPS):
        for r in prev:
            r.wait_recv()
        if c < N_HOPS - 1:
            prev = hop_rdmas(c)
            all_rdmas += prev
            for r in prev:
                r.start()
        else:
            hs = seq_q // 2
            sub_a = hop_rdmas(c, rows=slice(0, hs), idx_base=4 * c)
            sub_b = hop_rdmas(c, rows=slice(hs, seq_q), idx_base=4 * (c + 1))
            all_rdmas += sub_a + sub_b
            for r in sub_a + sub_b:
                r.start()
        compute_phase(c)
    for r in sub_a:
        r.wait_recv()
    compute_phase(N_DEV - 1, rows=slice(0, seq_q // 2), finalize=False)
    for r in sub_b:
        r.wait_recv()
    compute_phase(N_DEV - 1, rows=slice(seq_q // 2, seq_q), finalize=True)
    for r in all_rdmas:
        r.wait_send()


def kernel(Q, K, V):
    b, s, h, d = Q.shape
    scale = d**-0.5 * 1.4426950408889634
    q = (Q[0] * scale).astype(jnp.bfloat16).reshape(s, h * d)
    k = K[0].astype(jnp.bfloat16).reshape(s, h * d)
    v = V[0].astype(jnp.bfloat16).reshape(s, h * d)

    out = pl.pallas_call(
        _body,
        out_shape=jax.ShapeDtypeStruct((s, h * d), jnp.float32),
        in_specs=[pl.BlockSpec(memory_space=pltpu.VMEM)] * 3,
        out_specs=pl.BlockSpec(memory_space=pltpu.VMEM),
        scratch_shapes=[
            pltpu.VMEM((N_HOPS, s, h * d), jnp.bfloat16),
            pltpu.VMEM((N_HOPS, s, h * d), jnp.bfloat16),
            pltpu.VMEM((s, N_HEADS), jnp.float32),
            pltpu.VMEM((s, N_HEADS), jnp.float32),
            pltpu.SemaphoreType.DMA((4 * (N_HOPS + 1),)),
            pltpu.SemaphoreType.DMA((4 * (N_HOPS + 1),)),
        ],
        compiler_params=pltpu.CompilerParams(
            collective_id=0, vmem_limit_bytes=100 * 1024 * 1024
        ),
    )(q, k, v)
    return out.reshape(s, h, d)[None]
